# baseline (speedup 1.0000x reference)
import functools

import jax
import jax.numpy as jnp
from jax import lax
from jax.experimental import pallas as pl
from jax.experimental.pallas import tpu as pltpu
from jax.experimental.pallas import tpu_sc as plsc

_B, _H, _S, _D = 8, 16, 2048, 128
_L = 16
_HB = 8
_R = _B * _H * _S
_RV = _B * _H * _L

_BSC = 2
_NW = 32
_SLABS_PW = (_BSC * _H) // _NW
_ROWS_PW = _SLABS_PW * _S
_BLK = 256
_NBLK = _ROWS_PW // _BLK


def _tc_body(pos_ref, c_ref, v_ref, o_ref):
    o_ref[...] = c_ref[...]
    p0 = pos_ref[0]
    for h in range(_HB):
        o_ref[0, h, pl.ds(p0, _L), :] = v_ref[0, h, :, :]


def _tc_update_k(cache, pos, val):
    cache_spec = pl.BlockSpec((1, _HB, _S, _D), lambda i, j, p: (i, j, 0, 0))
    val_spec = pl.BlockSpec((1, _HB, _L, _D), lambda i, j, p: (i, j, 0, 0))
    return pl.pallas_call(
        _tc_body,
        grid_spec=pltpu.PrefetchScalarGridSpec(
            num_scalar_prefetch=1,
            grid=(_B, _H // _HB),
            in_specs=[cache_spec, val_spec],
            out_specs=cache_spec,
        ),
        out_shape=jax.ShapeDtypeStruct((_B, _H, _S, _D), jnp.float32),
        compiler_params=pltpu.CompilerParams(
            dimension_semantics=("arbitrary", "arbitrary"),
        ),
    )(pos, cache, val)


def _tc_fill_v_tail(v_partial2d, cache2d, pos, val2d):
    nj = _H // _HB
    read_spec = pl.BlockSpec(
        (_HB * _S, _D), lambda i, j, p: ((i + _BSC) * nj + j, 0))
    val_spec = pl.BlockSpec(
        (_HB * _L, _D), lambda i, j, p: ((i + _BSC) * nj + j, 0))
    alias_spec = pl.BlockSpec(memory_space=pltpu.HBM)

    def body(pos_ref, vp_ref, c_ref, v_ref, o_ref):
        o_ref[...] = c_ref[...]
        p0 = pos_ref[0]
        for h in range(_HB):
            o_ref[pl.ds(h * _S + p0, _L), :] = v_ref[pl.ds(h * _L, _L), :]

    return pl.pallas_call(
        body,
        grid_spec=pltpu.PrefetchScalarGridSpec(
            num_scalar_prefetch=1,
            grid=(_B - _BSC, nj),
            in_specs=[alias_spec, read_spec, val_spec],
            out_specs=read_spec,
        ),
        out_shape=jax.ShapeDtypeStruct((_R, _D), jnp.float32),
        input_output_aliases={1: 0},
        compiler_params=pltpu.CompilerParams(
            dimension_semantics=("arbitrary", "arbitrary"),
        ),
    )(pos, v_partial2d, cache2d, val2d)


def _sc_update_v_head(cache2d, pos, val2d):
    mesh = plsc.VectorSubcoreMesh(core_axis_name="c", subcore_axis_name="s")

    @functools.partial(
        pl.kernel, mesh=mesh,
        out_type=jax.ShapeDtypeStruct((_R, _D), jnp.float32),
        scratch_types=[
            pltpu.VMEM((_BLK, _D), jnp.float32),
            pltpu.VMEM((_BLK, _D), jnp.float32),
            pltpu.VMEM((_SLABS_PW * _L, _D), jnp.float32),
            pltpu.VMEM((_L,), jnp.int32),
            pltpu.VMEM((_SLABS_PW * _L,), jnp.int32),
            pltpu.SemaphoreType.DMA,
            pltpu.SemaphoreType.DMA,
            pltpu.SemaphoreType.DMA,
            pltpu.SemaphoreType.DMA,
            pltpu.SemaphoreType.DMA,
        ],
    )
    def k(pos_hbm, c_hbm, v_hbm, o_hbm,
          buf0, buf1, vbuf, pos_v, idx_v, rs0, rs1, ws0, ws1, vs):
        wid = lax.axis_index("s") * 2 + lax.axis_index("c")
        base = wid * _ROWS_PW
        vread = pltpu.make_async_copy(
            v_hbm.at[pl.ds(wid * (_SLABS_PW * _L), _SLABS_PW * _L)], vbuf, vs)
        vread.start()
        pltpu.sync_copy(pos_hbm, pos_v)
        pvec = pos_v[...]
        for t in range(_SLABS_PW):
            idx_v[pl.ds(t * _L, _L)] = pvec + (base + t * _S)

        bufs = (buf0, buf1)
        rsems = (rs0, rs1)
        wsems = (ws0, ws1)
        reads = [None, None]
        writes = [None, None]
        reads[0] = pltpu.make_async_copy(
            c_hbm.at[pl.ds(base, _BLK)], buf0, rs0)
        reads[0].start()
        for i in range(_NBLK):
            cur = i & 1
            nxt = 1 - cur
            if i + 1 < _NBLK:
                if writes[nxt] is not None:
                    writes[nxt].wait()
                    writes[nxt] = None
                reads[nxt] = pltpu.make_async_copy(
                    c_hbm.at[pl.ds(base + (i + 1) * _BLK, _BLK)],
                    bufs[nxt], rsems[nxt])
                reads[nxt].start()
            reads[cur].wait()
            writes[cur] = pltpu.make_async_copy(
                bufs[cur], o_hbm.at[pl.ds(base + i * _BLK, _BLK)], wsems[cur])
            writes[cur].start()
        for j in range(2):
            if writes[j] is not None:
                writes[j].wait()
        vread.wait()
        pltpu.sync_copy(vbuf, o_hbm.at[idx_v])

    return k(pos, cache2d, val2d)


def kernel(k_cache, v_cache, input_pos, k_val, v_val):
    k_new = _tc_update_k(k_cache, input_pos, k_val)
    vc2d = v_cache.reshape(_R, _D)
    vv2d = v_val.reshape(_RV, _D)
    v_head = _sc_update_v_head(vc2d, input_pos, vv2d)
    v_new = _tc_fill_v_tail(v_head, vc2d, input_pos, vv2d)
    return (k_new, v_new.reshape(_B, _H, _S, _D))


# speedup vs baseline: 1.1085x; 1.1085x over previous
import jax
import jax.numpy as jnp
from jax.experimental import pallas as pl
from jax.experimental.pallas import tpu as pltpu

_B, _H, _S, _D = 8, 16, 2048, 128
_L = 16


_HB = 4


def _body(pos_ref, kc_ref, vc_ref, kv_ref, vv_ref, ko_ref, vo_ref):
    ko_ref[...] = kc_ref[...]
    vo_ref[...] = vc_ref[...]
    p0 = pos_ref[0]
    for h in range(_HB):
        ko_ref[0, h, pl.ds(p0, _L), :] = kv_ref[0, h, :, :]
        vo_ref[0, h, pl.ds(p0, _L), :] = vv_ref[0, h, :, :]


def kernel(k_cache, v_cache, input_pos, k_val, v_val):
    cache_spec = pl.BlockSpec((1, _HB, _S, _D), lambda i, j, pos: (i, j, 0, 0))
    val_spec = pl.BlockSpec((1, _HB, _L, _D), lambda i, j, pos: (i, j, 0, 0))
    out = pl.pallas_call(
        _body,
        grid_spec=pltpu.PrefetchScalarGridSpec(
            num_scalar_prefetch=1,
            grid=(_B, _H // _HB),
            in_specs=[cache_spec, cache_spec, val_spec, val_spec],
            out_specs=[cache_spec, cache_spec],
        ),
        out_shape=[jax.ShapeDtypeStruct((_B, _H, _S, _D), jnp.float32)] * 2,
        compiler_params=pltpu.CompilerParams(
            dimension_semantics=("parallel", "parallel"),
        ),
    )(input_pos, k_cache, v_cache, k_val, v_val)
    return (out[0], out[1])
